# baseline (speedup 1.0000x reference)
import functools

import jax
import jax.numpy as jnp
from jax import lax
from jax.experimental import pallas as pl
from jax.experimental.pallas import tpu as pltpu
from jax.experimental.pallas import tpu_sc as plsc

_D = 64
_M = 4
_CHUNK = 128
_LANES = 16

_GDN = lax.GatherDimensionNumbers(
    offset_dims=(), collapsed_slice_dims=(0,), start_index_map=(0,))


def _xlane(v, perm):
    return lax.gather(v, perm[:, None], _GDN, (1,),
                      mode=lax.GatherScatterMode.PROMISE_IN_BOUNDS)


@functools.cache
def _make_sc_call(N, n_opc, n_opr, interpret=False):
    try:
        info = plsc.get_sparse_core_info()
        NC, NS = info.num_cores, info.num_subcores
    except ValueError:
        NC, NS = 2, 16
    NW = NC * NS
    assert N % (NW * _CHUNK) == 0
    per_w = N // NW
    n_chunks = per_w // _CHUNK
    CM = _CHUNK * _M

    mesh = plsc.VectorSubcoreMesh(
        core_axis_name="c", subcore_axis_name="s",
        num_cores=NC, num_subcores=NS)

    @functools.partial(
        pl.kernel,
        out_type=jax.ShapeDtypeStruct((N, _D), jnp.float32),
        mesh=mesh,
        interpret=interpret,
        compiler_params=pltpu.CompilerParams(use_tc_tiling_on_sc=False),
        scratch_types=[
            pltpu.VMEM((_CHUNK,), jnp.int32),
            pltpu.VMEM((CM,), jnp.int32),
            pltpu.VMEM((CM,), jnp.float32),
            pltpu.VMEM((CM, _D), jnp.float32),
            pltpu.VMEM((_CHUNK, _D), jnp.float32),
            pltpu.SemaphoreType.DMA,
            pltpu.SemaphoreType.DMA,
        ],
    )
    def sc_fn(opc_ids_hbm, opr_ids_hbm, opc_tab_hbm, opr_tab_hbm, out_hbm,
              opc_v, ids_v, w_v, rows_v, o_v, sem_ids, sem_g):
        wid = lax.axis_index("s") * NC + lax.axis_index("c")
        w_base = wid * per_w
        lane = lax.iota(jnp.int32, _LANES)
        perm1 = lane ^ 1
        perm2 = lane ^ 2

        def chunk_body(c, carry):
            base = w_base + c * _CHUNK
            cp0 = pltpu.async_copy(
                opc_ids_hbm.at[pl.ds(base, _CHUNK)], opc_v, sem_ids)
            cp1 = pltpu.async_copy(
                opr_ids_hbm.at[pl.ds(base * _M, CM)], ids_v, sem_ids)
            cp0.wait()
            cp1.wait()
            gs = [pltpu.async_copy(opc_tab_hbm.at[opc_v], o_v, sem_g)]
            for q in range(CM // _CHUNK):
                gs.append(pltpu.async_copy(
                    opr_tab_hbm.at[ids_v.at[pl.ds(q * _CHUNK, _CHUNK)]],
                    rows_v.at[pl.ds(q * _CHUNK, _CHUNK)], sem_g))
            for t in range(CM // _LANES):
                sl = pl.ds(t * _LANES, _LANES)
                mk = jnp.where(ids_v[sl] != 0, 1.0, 0.0)
                s = mk + _xlane(mk, perm1)
                s = s + _xlane(s, perm2)
                w_v[sl] = mk / (s + 1e-10)
            for g in gs:
                g.wait()

            def group_body(g, carry2):
                r0 = g * _LANES
                i0 = g * (_LANES // _M)
                wvec = w_v[pl.ds(r0, _LANES)]
                for j in range(_LANES // _M):
                    for dblk in range(_D // _LANES):
                        sl = pl.ds(dblk * _LANES, _LANES)
                        acc = wvec[4 * j] * rows_v[r0 + 4 * j, sl]
                        for m in range(1, _M):
                            acc = acc + wvec[4 * j + m] * rows_v[r0 + 4 * j + m, sl]
                        plsc.addupdate(o_v.at[i0 + j, sl], acc)
                return carry2

            lax.fori_loop(0, CM // _LANES, group_body, 0)
            pltpu.sync_copy(o_v, out_hbm.at[pl.ds(base, _CHUNK)])
            return carry

        lax.fori_loop(0, n_chunks, chunk_body, 0)

    return sc_fn


def kernel(opcode_ids, operand_ids, opcode_table, operand_table):
    B, S = opcode_ids.shape
    N = B * S
    opc_flat = opcode_ids.reshape(N).astype(jnp.int32)
    opr_flat = operand_ids.reshape(N * _M).astype(jnp.int32)
    fn = _make_sc_call(N, opcode_table.shape[0], operand_table.shape[0])
    out = fn(opc_flat, opr_flat, opcode_table, operand_table)
    return out.reshape(B, S, _D)
